# Initial kernel scaffold; baseline (speedup 1.0000x reference)
import jax
import jax.numpy as jnp
from jax.experimental import pallas as pl


def kernel(diag_seq, proc_seq, med_seq, c_embeddings, p_embeddings, m_embeddings, W_conv, b_conv, W_gat, att, b_gat, dia_med_hyperedge_attr, pro_med_hyperedge_attr):
    raise NotImplementedError("write your pallas kernel here")


# speedup vs baseline: 280.9183x; 280.9183x over previous
import functools

import jax
import jax.numpy as jnp
from jax import lax
from jax.experimental import pallas as pl
from jax.experimental.pallas import tpu as pltpu
from jax.experimental.pallas import tpu_sc as plsc

_D = 128


def _sc_gather(c_emb, p_emb, m_emb, diag_idx, proc_idx, med_idx):
    info = plsc.get_sparse_core_info()
    nc, ns = info.num_cores, info.num_subcores
    nw = nc * ns
    nd, npc, nm = diag_idx.shape[0], proc_idx.shape[0], med_idx.shape[0]
    bd, bp, bm = nd // nw, npc // nw, nm // nw

    mesh = plsc.VectorSubcoreMesh(core_axis_name="c", subcore_axis_name="s")

    @functools.partial(
        pl.kernel,
        mesh=mesh,
        out_type=[
            jax.ShapeDtypeStruct((nd, _D), jnp.float32),
            jax.ShapeDtypeStruct((npc, _D), jnp.float32),
            jax.ShapeDtypeStruct((nm, _D), jnp.float32),
        ],
        scratch_types=[
            pltpu.VMEM((bd,), jnp.int32),
            pltpu.VMEM((bp,), jnp.int32),
            pltpu.VMEM((bm,), jnp.int32),
            pltpu.VMEM((bd, _D), jnp.float32),
            pltpu.VMEM((bp, _D), jnp.float32),
            pltpu.VMEM((bm, _D), jnp.float32),
            pltpu.SemaphoreType.DMA,
            pltpu.SemaphoreType.DMA,
            pltpu.SemaphoreType.DMA,
        ],
    )
    def gather_kernel(c_hbm, p_hbm, m_hbm, di_hbm, pi_hbm, mi_hbm,
                      od_hbm, op_hbm, om_hbm,
                      di_v, pi_v, mi_v, dr_v, pr_v, mr_v, sd, sp, sm):
        wid = lax.axis_index("s") * nc + lax.axis_index("c")
        pltpu.sync_copy(di_hbm.at[pl.ds(wid * bd, bd)], di_v)
        pltpu.sync_copy(pi_hbm.at[pl.ds(wid * bp, bp)], pi_v)
        pltpu.sync_copy(mi_hbm.at[pl.ds(wid * bm, bm)], mi_v)
        cd = pltpu.async_copy(c_hbm.at[di_v], dr_v, sd)
        cp = pltpu.async_copy(p_hbm.at[pi_v], pr_v, sp)
        cm = pltpu.async_copy(m_hbm.at[mi_v], mr_v, sm)
        cd.wait()
        cp.wait()
        cm.wait()
        pltpu.sync_copy(dr_v, od_hbm.at[pl.ds(wid * bd, bd)])
        pltpu.sync_copy(pr_v, op_hbm.at[pl.ds(wid * bp, bp)])
        pltpu.sync_copy(mr_v, om_hbm.at[pl.ds(wid * bm, bm)])

    return gather_kernel(c_emb, p_emb, m_emb, diag_idx, proc_idx, med_idx)


def _dense_body(xd_ref, xp_ref, xm_ref, wc_ref, bc_ref, wg_ref,
                a1_ref, a2_ref, bg_ref, hed_ref, hep_ref, out_ref):
    xd = xd_ref[...]
    xp = xp_ref[...]
    xm = xm_ref[...]
    wc = wc_ref[...]
    bc = bc_ref[...]
    wg = wg_ref[...]
    a1 = a1_ref[...]
    a2 = a2_ref[...]
    bg = bg_ref[...]

    nd = xd.shape[0]
    npc = xp.shape[0]
    nm = xm.shape[0]

    md = jnp.mean(xd, axis=0, keepdims=True) @ wc + bc
    mp = jnp.mean(xp, axis=0, keepdims=True) @ wc + bc
    mm = jnp.mean(xm, axis=0, keepdims=True) @ wc + bc

    xtm = xm @ wg
    um_row = lax.dot_general(a1, xtm, (((0,), (1,)), ((), ())))

    def dual_att(x_nodes, he_attr, n_edges):
        xt = x_nodes @ wg
        ea = he_attr @ wg
        u = xt @ a1
        v = ea @ a2

        def leaky(z):
            return jnp.where(z >= 0.0, z, 0.2 * z)

        l_self = leaky(u + v)
        logits = leaky(um_row + v)
        m = jnp.maximum(jnp.max(logits, axis=1, keepdims=True), l_self)
        e = jnp.exp(logits - m)
        e_self = jnp.exp(l_self - m)
        r = 1.0 / (jnp.sum(e, axis=1, keepdims=True) + e_self + 1e-16)
        att_m = e * r
        att_self = e_self * r

        binv = 1.0 / (nm + 1.0)
        ef = binv * (att_self * xt + att_m @ xtm)
        out_nodes = att_self * ef + bg
        out_meds = (1.0 / n_edges) * lax.dot_general(
            att_m, ef, (((0,), (0,)), ((), ()))) + bg
        return out_nodes, out_meds

    dia_nodes, dia_meds = dual_att(xd, hed_ref[...], nd)
    pro_nodes, pro_meds = dual_att(xp, hep_ref[...], npc)

    out_ref[0:nd, 0:_D] = dia_nodes
    out_ref[0:nd, _D:2 * _D] = jnp.broadcast_to(md, (nd, _D))
    out_ref[nd:nd + nm, 0:_D] = dia_meds
    out_ref[nd:nd + nm, _D:2 * _D] = jnp.broadcast_to(mm, (nm, _D))
    r2 = nd + nm
    out_ref[r2:r2 + npc, 0:_D] = pro_nodes
    out_ref[r2:r2 + npc, _D:2 * _D] = jnp.broadcast_to(mp, (npc, _D))
    r3 = r2 + npc
    out_ref[r3:r3 + nm, 0:_D] = pro_meds
    out_ref[r3:r3 + nm, _D:2 * _D] = jnp.broadcast_to(mm, (nm, _D))


def _tc_dense(xd, xp, xm, w_conv, b_conv, w_gat, att, b_gat, hed, hep):
    nd, npc, nm = xd.shape[0], xp.shape[0], xm.shape[0]
    n_out = nd + nm + npc + nm
    return pl.pallas_call(
        _dense_body,
        out_shape=jax.ShapeDtypeStruct((n_out, 2 * _D), jnp.float32),
    )(xd, xp, xm,
      w_conv, b_conv.reshape(1, _D), w_gat,
      att[:_D].reshape(_D, 1), att[_D:].reshape(_D, 1),
      b_gat.reshape(1, _D), hed, hep)


def kernel(diag_seq, proc_seq, med_seq, c_embeddings, p_embeddings,
           m_embeddings, W_conv, b_conv, W_gat, att, b_gat,
           dia_med_hyperedge_attr, pro_med_hyperedge_attr):
    xd, xp, xm = _sc_gather(
        c_embeddings, p_embeddings, m_embeddings,
        diag_seq.astype(jnp.int32), proc_seq.astype(jnp.int32),
        med_seq.astype(jnp.int32))
    return _tc_dense(xd, xp, xm, W_conv, b_conv, W_gat, att, b_gat,
                     dia_med_hyperedge_attr, pro_med_hyperedge_attr)


# speedup vs baseline: 297.1779x; 1.0579x over previous
import functools

import jax
import jax.numpy as jnp
from jax import lax
from jax.experimental import pallas as pl
from jax.experimental.pallas import tpu as pltpu
from jax.experimental.pallas import tpu_sc as plsc

_D = 128
_ND = 512
_NP = 512
_NM = 256


def _sc_gather(c_emb, p_emb, m_emb, diag_idx, proc_idx, med_idx):
    info = plsc.get_sparse_core_info()
    nc, ns = info.num_cores, info.num_subcores
    nw = nc * ns
    bd, bp, bm = _ND // nw, _NP // nw, _NM // nw

    mesh = plsc.VectorSubcoreMesh(core_axis_name="c", subcore_axis_name="s")

    @functools.partial(
        pl.kernel,
        mesh=mesh,
        out_type=jax.ShapeDtypeStruct((_ND + _NP + _NM, _D), jnp.float32),
        scratch_types=[
            pltpu.VMEM((bd,), jnp.int32),
            pltpu.VMEM((bp,), jnp.int32),
            pltpu.VMEM((bm,), jnp.int32),
            pltpu.VMEM((bd, _D), jnp.float32),
            pltpu.VMEM((bp, _D), jnp.float32),
            pltpu.VMEM((bm, _D), jnp.float32),
            pltpu.SemaphoreType.DMA,
            pltpu.SemaphoreType.DMA,
            pltpu.SemaphoreType.DMA,
        ],
    )
    def gather_kernel(c_hbm, p_hbm, m_hbm, di_hbm, pi_hbm, mi_hbm, o_hbm,
                      di_v, pi_v, mi_v, dr_v, pr_v, mr_v, s0, s1, s2):
        wid = lax.axis_index("s") * nc + lax.axis_index("c")
        i0 = pltpu.async_copy(di_hbm.at[pl.ds(wid * bd, bd)], di_v, s0)
        i1 = pltpu.async_copy(pi_hbm.at[pl.ds(wid * bp, bp)], pi_v, s1)
        i2 = pltpu.async_copy(mi_hbm.at[pl.ds(wid * bm, bm)], mi_v, s2)
        i0.wait()
        g0 = pltpu.async_copy(c_hbm.at[di_v], dr_v, s0)
        i1.wait()
        g1 = pltpu.async_copy(p_hbm.at[pi_v], pr_v, s1)
        i2.wait()
        g2 = pltpu.async_copy(m_hbm.at[mi_v], mr_v, s2)
        g0.wait()
        w0 = pltpu.async_copy(dr_v, o_hbm.at[pl.ds(wid * bd, bd)], s0)
        g1.wait()
        w1 = pltpu.async_copy(pr_v, o_hbm.at[pl.ds(_ND + wid * bp, bp)], s1)
        g2.wait()
        w2 = pltpu.async_copy(mr_v, o_hbm.at[pl.ds(_ND + _NP + wid * bm, bm)],
                              s2)
        w0.wait()
        w1.wait()
        w2.wait()

    return gather_kernel(c_emb, p_emb, m_emb, diag_idx, proc_idx, med_idx)


def _dense_body(g_ref, wc_ref, bc_ref, wg_ref, att_ref, bg_ref,
                hed_ref, hep_ref, out_ref):
    xd = g_ref[0:_ND, :]
    xp = g_ref[_ND:_ND + _NP, :]
    xm = g_ref[_ND + _NP:_ND + _NP + _NM, :]
    wc = wc_ref[...]
    bc = bc_ref[...]
    wg = wg_ref[...]
    a1 = att_ref[0:1, :]
    a2 = att_ref[1:2, :]
    bg = bg_ref[...]

    md = jnp.mean(xd, axis=0, keepdims=True) @ wc + bc
    mp = jnp.mean(xp, axis=0, keepdims=True) @ wc + bc
    mm = jnp.mean(xm, axis=0, keepdims=True) @ wc + bc

    xtm = xm @ wg
    um_row = lax.dot_general(a1, xtm, (((1,), (1,)), ((), ())))

    def dual_att(x_nodes, he_attr, n_edges):
        xt = x_nodes @ wg
        ea = he_attr @ wg
        u = lax.dot_general(xt, a1, (((1,), (1,)), ((), ())))
        v = lax.dot_general(ea, a2, (((1,), (1,)), ((), ())))

        def leaky(z):
            return jnp.where(z >= 0.0, z, 0.2 * z)

        l_self = leaky(u + v)
        logits = leaky(um_row + v)
        m = jnp.maximum(jnp.max(logits, axis=1, keepdims=True), l_self)
        e = jnp.exp(logits - m)
        e_self = jnp.exp(l_self - m)
        r = 1.0 / (jnp.sum(e, axis=1, keepdims=True) + e_self + 1e-16)
        att_m = e * r
        att_self = e_self * r

        binv = 1.0 / (_NM + 1.0)
        ef = binv * (att_self * xt + att_m @ xtm)
        out_nodes = att_self * ef + bg
        out_meds = (1.0 / n_edges) * lax.dot_general(
            att_m, ef, (((0,), (0,)), ((), ()))) + bg
        return out_nodes, out_meds

    dia_nodes, dia_meds = dual_att(xd, hed_ref[...], _ND)
    pro_nodes, pro_meds = dual_att(xp, hep_ref[...], _NP)

    out_ref[0:_ND, 0:_D] = dia_nodes
    out_ref[0:_ND, _D:2 * _D] = jnp.broadcast_to(md, (_ND, _D))
    out_ref[_ND:_ND + _NM, 0:_D] = dia_meds
    out_ref[_ND:_ND + _NM, _D:2 * _D] = jnp.broadcast_to(mm, (_NM, _D))
    r2 = _ND + _NM
    out_ref[r2:r2 + _NP, 0:_D] = pro_nodes
    out_ref[r2:r2 + _NP, _D:2 * _D] = jnp.broadcast_to(mp, (_NP, _D))
    r3 = r2 + _NP
    out_ref[r3:r3 + _NM, 0:_D] = pro_meds
    out_ref[r3:r3 + _NM, _D:2 * _D] = jnp.broadcast_to(mm, (_NM, _D))


def kernel(diag_seq, proc_seq, med_seq, c_embeddings, p_embeddings,
           m_embeddings, W_conv, b_conv, W_gat, att, b_gat,
           dia_med_hyperedge_attr, pro_med_hyperedge_attr):
    gathered = _sc_gather(
        c_embeddings, p_embeddings, m_embeddings,
        diag_seq.astype(jnp.int32), proc_seq.astype(jnp.int32),
        med_seq.astype(jnp.int32))
    n_out = _ND + _NP + 2 * _NM
    return pl.pallas_call(
        _dense_body,
        out_shape=jax.ShapeDtypeStruct((n_out, 2 * _D), jnp.float32),
    )(gathered, W_conv, b_conv.reshape(1, _D), W_gat,
      att.reshape(2, _D), b_gat.reshape(1, _D),
      dia_med_hyperedge_attr, pro_med_hyperedge_attr)
